# baseline (speedup 1.0000x reference)
import functools

import jax
import jax.numpy as jnp
from jax import lax
from jax.experimental import pallas as pl
from jax.experimental.pallas import tpu as pltpu
from jax.experimental.pallas import tpu_sc as plsc

B = 16384
NUM_DENSE = 13
NUM_SPARSE = 26
VOCAB = 100000
D = 64
NF = NUM_SPARSE + 1
NPAIR = NF * (NF - 1) // 2
INT_DIM = D + NPAIR

GATHER_WINDOW = 128
N_IDX = B * NUM_SPARSE

BB = 256


N_WORKERS = 32
PER_W = N_IDX // N_WORKERS
CHUNK = 1024


def _sc_gather(emb_flat, flat_idx):
    mesh = plsc.VectorSubcoreMesh(core_axis_name="core", subcore_axis_name="subcore")

    @functools.partial(
        pl.kernel,
        out_type=jax.ShapeDtypeStruct((N_IDX, D), jnp.float32),
        mesh=mesh,
        scratch_types=[pltpu.VMEM((CHUNK,), jnp.int32), pltpu.SemaphoreType.DMA],
    )
    def k(x_hbm, i_hbm, o_hbm, idx_vmem, sem):
        wid = lax.axis_index("subcore") * 2 + lax.axis_index("core")
        base = wid * PER_W

        @pl.loop(0, PER_W // CHUNK)
        def _(c):
            cbase = base + c * CHUNK
            pltpu.sync_copy(i_hbm.at[pl.ds(cbase, CHUNK)], idx_vmem)

            @pl.loop(0, CHUNK, step=16)
            def _(j):
                v = idx_vmem[pl.ds(j, 16)]
                for t in range(16):
                    pltpu.make_async_copy(
                        x_hbm.at[pl.ds(v[t], 1), :],
                        o_hbm.at[pl.ds(cbase + j + t, 1), :],
                        sem,
                    ).start()

            @pl.loop(0, CHUNK)
            def _(j):
                pltpu.make_async_copy(
                    x_hbm.at[pl.ds(0, 1), :],
                    o_hbm.at[pl.ds(0, 1), :],
                    sem,
                ).wait()

    return k(emb_flat, flat_idx)


def _tc_body(numT_ref, emb_ref,
             bw0T, bb0, bw1, bb1, bw2, bb2,
             tw0T, tb0, tw1, tb1, tw2, tb2, tw3, tb3, tw4, tb4,
             out_ref):
    f32 = jnp.float32
    xT = numT_ref[...]
    h = jnp.maximum(jnp.dot(bw0T[...], xT, preferred_element_type=f32) + bb0[...], 0.0)
    h = jnp.maximum(jnp.dot(bw1[...], h, preferred_element_type=f32) + bb1[...], 0.0)
    bot = jnp.maximum(jnp.dot(bw2[...], h, preferred_element_type=f32) + bb2[...], 0.0)

    embT = jnp.transpose(emb_ref[...], (0, 2, 1))
    T3 = jnp.concatenate([bot[None], embT], axis=0)
    zparts = []
    for i in range(1, NF):
        prod = T3[:i] * T3[i][None]
        zparts.append(jnp.sum(prod, axis=1))
    zcat = jnp.concatenate(zparts, axis=0)
    topT = jnp.concatenate([bot, zcat], axis=0)

    y = jnp.maximum(jnp.dot(tw0T[...], topT, preferred_element_type=f32) + tb0[...], 0.0)
    y = jnp.maximum(jnp.dot(tw1[...], y, preferred_element_type=f32) + tb1[...], 0.0)
    y = jnp.maximum(jnp.dot(tw2[...], y, preferred_element_type=f32) + tb2[...], 0.0)
    y = jnp.maximum(jnp.dot(tw3[...], y, preferred_element_type=f32) + tb3[...], 0.0)
    out_ref[...] = jnp.dot(tw4[...], y, preferred_element_type=f32) + tb4[...]


def _tc_forward(numT, emb_fm, weightsT, interpret=False):
    full = lambda a: pl.BlockSpec(a.shape, lambda b: tuple(0 for _ in a.shape))
    in_specs = [
        pl.BlockSpec((NUM_DENSE, BB), lambda b: (0, b)),
        pl.BlockSpec((NUM_SPARSE, BB, D), lambda b: (0, b, 0)),
    ] + [full(w) for w in weightsT]
    out = pl.pallas_call(
        _tc_body,
        grid=(B // BB,),
        in_specs=in_specs,
        out_specs=pl.BlockSpec((1, BB), lambda b: (0, b)),
        out_shape=jax.ShapeDtypeStruct((1, B), jnp.float32),
        interpret=interpret,
    )(numT, emb_fm, *weightsT)
    return out.reshape(B)


def kernel(numerical_input, categorical_input, emb_tables,
           bw0, bb0, bw1, bb1, bw2, bb2,
           tw0, tb0, tw1, tb1, tw2, tb2, tw3, tb3, tw4, tb4):
    emb_flat = emb_tables.reshape(NUM_SPARSE * VOCAB, D)
    offs = (jnp.arange(NUM_SPARSE, dtype=jnp.int32) * VOCAB)[:, None]
    flat_idx = (categorical_input.T.astype(jnp.int32) + offs).reshape(N_IDX)
    gathered = _sc_gather(emb_flat, flat_idx)
    emb_fm = gathered.reshape(NUM_SPARSE, B, D)

    numT = numerical_input.T
    col = lambda v: v.reshape(-1, 1)
    weightsT = [
        bw0.T, col(bb0), bw1.T, col(bb1), bw2.T, col(bb2),
        tw0.T, col(tb0), tw1.T, col(tb1), tw2.T, col(tb2),
        tw3.T, col(tb3), tw4.T, col(tb4),
    ]
    return _tc_forward(numT, emb_fm, weightsT)


# speedup vs baseline: 4.0351x; 4.0351x over previous
import functools

import jax
import jax.numpy as jnp
from jax.experimental import pallas as pl
from jax.experimental.pallas import tpu as pltpu
from jax.experimental.pallas import tpu_sc as plsc

B = 16384
NUM_DENSE = 13
NUM_SPARSE = 26
VOCAB = 100000
D = 64
NF = NUM_SPARSE + 1
NPAIR = NF * (NF - 1) // 2
INT_DIM = D + NPAIR

N_ROWS = NUM_SPARSE * VOCAB
N_PACK = N_ROWS // 2
GATHER_WINDOW = 128
N_IDX = B * NUM_SPARSE

BB = 256

REPACK_BLK = 4000


def _repack_body(x_ref, o_ref):
    h = REPACK_BLK // 2
    o_ref[:, :D] = x_ref[:h]
    o_ref[:, D:] = x_ref[h:]


def _tc_repack(emb_flat):
    return pl.pallas_call(
        _repack_body,
        grid=(N_ROWS // REPACK_BLK,),
        in_specs=[pl.BlockSpec((REPACK_BLK, D), lambda i: (i, 0))],
        out_specs=pl.BlockSpec((REPACK_BLK // 2, 2 * D), lambda i: (i, 0)),
        out_shape=jax.ShapeDtypeStruct((N_PACK, 2 * D), jnp.float32),
    )(emb_flat)


def _sc_gather(packed, pair_idx):
    mesh = plsc.VectorSubcoreMesh(core_axis_name="core", subcore_axis_name="subcore")

    @functools.partial(
        pl.kernel,
        out_type=jax.ShapeDtypeStruct((N_IDX, 2 * D), jnp.float32),
        mesh=mesh,
    )
    def k(x_hbm, i_hbm, o_hbm):
        def body(i_vmem, o_vmem):
            pltpu.sync_copy(x_hbm.at[i_vmem.at[0]], o_vmem)

        pltpu.emit_pipeline(
            body,
            grid=(N_IDX // GATHER_WINDOW,),
            in_specs=[pl.BlockSpec((1, GATHER_WINDOW), lambda i: (0, i))],
            out_specs=[pl.BlockSpec((GATHER_WINDOW, 2 * D), lambda i: (i, 0))],
            core_axis_name=("core", "subcore"),
            dimension_semantics=(pltpu.PARALLEL,),
        )(i_hbm, o_hbm)

    return k(packed, pair_idx)


def _tc_body(numT_ref, emb_ref, par_ref,
             bw0T, bb0, bw1, bb1, bw2, bb2,
             tw0T, tb0, tw1, tb1, tw2, tb2, tw3, tb3, tw4, tb4,
             out_ref):
    f32 = jnp.float32
    xT = numT_ref[...]
    h = jnp.maximum(jnp.dot(bw0T[...], xT, preferred_element_type=f32) + bb0[...], 0.0)
    h = jnp.maximum(jnp.dot(bw1[...], h, preferred_element_type=f32) + bb1[...], 0.0)
    bot = jnp.maximum(jnp.dot(bw2[...], h, preferred_element_type=f32) + bb2[...], 0.0)

    Ts = [bot]
    for f in range(NUM_SPARSE):
        gT = emb_ref[f].T
        m = par_ref[f]
        Ts.append(gT[:D] + m * (gT[D:] - gT[:D]))
    T3 = jnp.stack(Ts, axis=0)
    zparts = []
    for i in range(1, NF):
        prod = T3[:i] * T3[i][None]
        zparts.append(jnp.sum(prod, axis=1))
    zcat = jnp.concatenate(zparts, axis=0)
    topT = jnp.concatenate([bot, zcat], axis=0)

    y = jnp.maximum(jnp.dot(tw0T[...], topT, preferred_element_type=f32) + tb0[...], 0.0)
    y = jnp.maximum(jnp.dot(tw1[...], y, preferred_element_type=f32) + tb1[...], 0.0)
    y = jnp.maximum(jnp.dot(tw2[...], y, preferred_element_type=f32) + tb2[...], 0.0)
    y = jnp.maximum(jnp.dot(tw3[...], y, preferred_element_type=f32) + tb3[...], 0.0)
    out_ref[...] = jnp.dot(tw4[...], y, preferred_element_type=f32) + tb4[...]


def _tc_forward(numT, emb_pk, parity, weightsT, interpret=False):
    full = lambda a: pl.BlockSpec(a.shape, lambda b: tuple(0 for _ in a.shape))
    in_specs = [
        pl.BlockSpec((NUM_DENSE, BB), lambda b: (0, b)),
        pl.BlockSpec((NUM_SPARSE, BB, 2 * D), lambda b: (0, b, 0)),
        pl.BlockSpec((NUM_SPARSE, 1, BB), lambda b: (0, 0, b)),
    ] + [full(w) for w in weightsT]
    out = pl.pallas_call(
        _tc_body,
        grid=(B // BB,),
        in_specs=in_specs,
        out_specs=pl.BlockSpec((1, BB), lambda b: (0, b)),
        out_shape=jax.ShapeDtypeStruct((1, B), jnp.float32),
        interpret=interpret,
    )(numT, emb_pk, parity, *weightsT)
    return out.reshape(B)


def kernel(numerical_input, categorical_input, emb_tables,
           bw0, bb0, bw1, bb1, bw2, bb2,
           tw0, tb0, tw1, tb1, tw2, tb2, tw3, tb3, tw4, tb4):
    emb_flat = emb_tables.reshape(N_ROWS, D)
    packed = _tc_repack(emb_flat)
    offs = (jnp.arange(NUM_SPARSE, dtype=jnp.int32) * VOCAB)[:, None]
    flat_idx = categorical_input.T.astype(jnp.int32) + offs
    h = REPACK_BLK // 2
    blk_i = flat_idx // REPACK_BLK
    rem = flat_idx % REPACK_BLK
    pair_idx = (blk_i * h + rem % h).reshape(1, N_IDX)
    parity = (rem >= h).astype(jnp.float32).reshape(NUM_SPARSE, 1, B)
    gathered = _sc_gather(packed, pair_idx)
    emb_pk = gathered.reshape(NUM_SPARSE, B, 2 * D)

    numT = numerical_input.T
    col = lambda v: v.reshape(-1, 1)
    weightsT = [
        bw0.T, col(bb0), bw1.T, col(bb1), bw2.T, col(bb2),
        tw0.T, col(tb0), tw1.T, col(tb1), tw2.T, col(tb2),
        tw3.T, col(tb3), tw4.T, col(tb4),
    ]
    return _tc_forward(numT, emb_pk, parity, weightsT)


# speedup vs baseline: 4.1519x; 1.0289x over previous
import functools

import jax
import jax.numpy as jnp
from jax.experimental import pallas as pl
from jax.experimental.pallas import tpu as pltpu
from jax.experimental.pallas import tpu_sc as plsc

B = 16384
NUM_DENSE = 13
NUM_SPARSE = 26
VOCAB = 100000
D = 64
NF = NUM_SPARSE + 1
NPAIR = NF * (NF - 1) // 2
INT_DIM = D + NPAIR

N_ROWS = NUM_SPARSE * VOCAB
N_PACK = N_ROWS // 2
GATHER_WINDOW = 128
N_IDX = B * NUM_SPARSE

BB = 256

REPACK_BLK = 8000
QROWS = REPACK_BLK // 4


def _repack_body(x_ref, o_ref):
    xi = jax.lax.bitcast_convert_type(x_ref[...], jnp.int32)
    xr = jnp.bitwise_and(xi + 0x8000, jnp.int32(-65536))
    hi = xr[:, : D // 2]
    lo = jax.lax.shift_right_logical(xr[:, D // 2:], 16)
    p = jax.lax.bitcast_convert_type(jnp.bitwise_or(hi, lo), jnp.float32)
    for q in range(4):
        o_ref[:, 32 * q:32 * (q + 1)] = p[QROWS * q:QROWS * (q + 1)]


def _tc_repack(emb_flat):
    return pl.pallas_call(
        _repack_body,
        grid=(N_ROWS // REPACK_BLK,),
        in_specs=[pl.BlockSpec((REPACK_BLK, D), lambda i: (i, 0))],
        out_specs=pl.BlockSpec((QROWS, 2 * D), lambda i: (i, 0)),
        out_shape=jax.ShapeDtypeStruct((N_ROWS // 4, 2 * D), jnp.float32),
    )(emb_flat)


def _sc_gather(packed, pair_idx):
    mesh = plsc.VectorSubcoreMesh(core_axis_name="core", subcore_axis_name="subcore")

    @functools.partial(
        pl.kernel,
        out_type=jax.ShapeDtypeStruct((N_IDX, 2 * D), jnp.float32),
        mesh=mesh,
    )
    def k(x_hbm, i_hbm, o_hbm):
        def body(i_vmem, o_vmem):
            pltpu.sync_copy(x_hbm.at[i_vmem.at[0]], o_vmem)

        pltpu.emit_pipeline(
            body,
            grid=(N_IDX // GATHER_WINDOW,),
            in_specs=[pl.BlockSpec((1, GATHER_WINDOW), lambda i: (0, i))],
            out_specs=[pl.BlockSpec((GATHER_WINDOW, 2 * D), lambda i: (i, 0))],
            core_axis_name=("core", "subcore"),
            dimension_semantics=(pltpu.PARALLEL,),
        )(i_hbm, o_hbm)

    return k(packed, pair_idx)


def _tc_body(numT_ref, emb_ref, par_ref,
             bw0T, bb0, bw1, bb1, bw2, bb2,
             tw0T, tb0, tw1, tb1, tw2, tb2, tw3, tb3, tw4, tb4,
             out_ref):
    f32 = jnp.float32
    xT = numT_ref[...]
    h = jnp.maximum(jnp.dot(bw0T[...], xT, preferred_element_type=f32) + bb0[...], 0.0)
    h = jnp.maximum(jnp.dot(bw1[...], h, preferred_element_type=f32) + bb1[...], 0.0)
    bot = jnp.maximum(jnp.dot(bw2[...], h, preferred_element_type=f32) + bb2[...], 0.0)

    Ts = [bot]
    for f in range(NUM_SPARSE):
        gT = emb_ref[f].T
        qv = par_ref[f]
        s01 = jnp.where(qv < 0.5, gT[0:32], gT[32:64])
        s23 = jnp.where(qv < 2.5, gT[64:96], gT[96:128])
        gsel = jnp.where(qv < 1.5, s01, s23)
        gi = jax.lax.bitcast_convert_type(gsel, jnp.int32)
        ehi = jax.lax.bitcast_convert_type(
            jnp.bitwise_and(gi, jnp.int32(-65536)), jnp.float32)
        elo = jax.lax.bitcast_convert_type(
            jax.lax.shift_left(gi, 16), jnp.float32)
        Ts.append(jnp.concatenate([ehi, elo], axis=0))
    T3 = jnp.stack(Ts, axis=0)
    zparts = []
    for i in range(1, NF):
        prod = T3[:i] * T3[i][None]
        zparts.append(jnp.sum(prod, axis=1))
    zcat = jnp.concatenate(zparts, axis=0)
    topT = jnp.concatenate([bot, zcat], axis=0)

    y = jnp.maximum(jnp.dot(tw0T[...], topT, preferred_element_type=f32) + tb0[...], 0.0)
    y = jnp.maximum(jnp.dot(tw1[...], y, preferred_element_type=f32) + tb1[...], 0.0)
    y = jnp.maximum(jnp.dot(tw2[...], y, preferred_element_type=f32) + tb2[...], 0.0)
    y = jnp.maximum(jnp.dot(tw3[...], y, preferred_element_type=f32) + tb3[...], 0.0)
    out_ref[...] = jnp.dot(tw4[...], y, preferred_element_type=f32) + tb4[...]


def _tc_forward(numT, emb_pk, parity, weightsT, interpret=False):
    full = lambda a: pl.BlockSpec(a.shape, lambda b: tuple(0 for _ in a.shape))
    in_specs = [
        pl.BlockSpec((NUM_DENSE, BB), lambda b: (0, b)),
        pl.BlockSpec((NUM_SPARSE, BB, 2 * D), lambda b: (0, b, 0)),
        pl.BlockSpec((NUM_SPARSE, 1, BB), lambda b: (0, 0, b)),
    ] + [full(w) for w in weightsT]
    out = pl.pallas_call(
        _tc_body,
        grid=(B // BB,),
        in_specs=in_specs,
        out_specs=pl.BlockSpec((1, BB), lambda b: (0, b)),
        out_shape=jax.ShapeDtypeStruct((1, B), jnp.float32),
        interpret=interpret,
    )(numT, emb_pk, parity, *weightsT)
    return out.reshape(B)


def kernel(numerical_input, categorical_input, emb_tables,
           bw0, bb0, bw1, bb1, bw2, bb2,
           tw0, tb0, tw1, tb1, tw2, tb2, tw3, tb3, tw4, tb4):
    emb_flat = emb_tables.reshape(N_ROWS, D)
    packed = _tc_repack(emb_flat)
    offs = (jnp.arange(NUM_SPARSE, dtype=jnp.int32) * VOCAB)[:, None]
    flat_idx = categorical_input.T.astype(jnp.int32) + offs
    blk_i = flat_idx // REPACK_BLK
    rem = flat_idx % REPACK_BLK
    pair_idx = (blk_i * QROWS + rem % QROWS).reshape(1, N_IDX)
    parity = (rem // QROWS).astype(jnp.float32).reshape(NUM_SPARSE, 1, B)
    gathered = _sc_gather(packed, pair_idx)
    emb_pk = gathered.reshape(NUM_SPARSE, B, 2 * D)

    numT = numerical_input.T
    col = lambda v: v.reshape(-1, 1)
    weightsT = [
        bw0.T, col(bb0), bw1.T, col(bb1), bw2.T, col(bb2),
        tw0.T, col(tb0), tw1.T, col(tb1), tw2.T, col(tb2),
        tw3.T, col(tb3), tw4.T, col(tb4),
    ]
    return _tc_forward(numT, emb_pk, parity, weightsT)
